# baseline (speedup 1.0000x reference)
import functools

import jax
import jax.numpy as jnp
from jax import lax
from jax.experimental import pallas as pl
from jax.experimental.pallas import tpu as pltpu
from jax.experimental.pallas import tpu_sc as plsc

E = 8
R = 64
D = 2048
ER = E * R
SCALING = 4.0

BM = 1024
SUB = 4

T_TOK = 8192
NW = 32
B_PER_W = T_TOK // NW
CHUNK = 32


def _make_sc_gather():
    mesh = plsc.VectorSubcoreMesh(core_axis_name="c", subcore_axis_name="s")

    @functools.partial(
        pl.kernel, mesh=mesh,
        out_type=jax.ShapeDtypeStruct((T_TOK, D), jnp.float32),
        scratch_types=[
            pltpu.VMEM((CHUNK,), jnp.int32),
            pltpu.VMEM((CHUNK, D), jnp.float32),
            pltpu.SemaphoreType.DMA,
        ],
    )
    def gather_rows(x_hbm, idx_hbm, out_hbm, idx_v, rows_v, sem):
        wid = lax.axis_index("s") * 2 + lax.axis_index("c")
        for j in range(B_PER_W // CHUNK):
            base = wid * B_PER_W + j * CHUNK
            pltpu.sync_copy(idx_hbm.at[pl.ds(base, CHUNK)], idx_v)
            pltpu.async_copy(x_hbm.at[idx_v], rows_v, sem).wait()
            pltpu.sync_copy(rows_v, out_hbm.at[pl.ds(base, CHUNK)])

    return gather_rows


_sc_gather = _make_sc_gather()


def _fused_dense_kernel(x_ref, wg_ref, wa_ref, wb_ref, o_ref):
    wg = wg_ref[...]
    wa = wa_ref[...]
    wb = wb_ref[...]
    hb = BM // SUB
    xs = [x_ref[s * hb:(s + 1) * hb, :] for s in range(SUB)]
    logits = [jax.lax.dot_general(x, wg, (((1,), (1,)), ((), ())),
                                  preferred_element_type=jnp.float32)
              for x in xs]
    cmats = []
    for lg in logits:
        maxv = jnp.max(lg, axis=1, keepdims=True)
        denom = jnp.sum(jnp.exp(lg - maxv), axis=1, keepdims=True)
        coef = SCALING / denom
        eidx = jax.lax.broadcasted_iota(jnp.int32, lg.shape, 1)
        gate = jnp.min(jnp.where(lg >= maxv, eidx, E), axis=1, keepdims=True)
        col_e = jax.lax.broadcasted_iota(jnp.int32, (hb, ER), 1) // R
        cmats.append(jnp.where(col_e == gate, coef, 0.0))
    hs = [jax.lax.dot_general(x, wa, (((1,), (1,)), ((), ())),
                              preferred_element_type=jnp.float32)
          for x in xs]
    for s in range(SUB):
        o_ref[s * hb:(s + 1) * hb, :] = jax.lax.dot_general(
            hs[s] * cmats[s], wb, (((1,), (0,)), ((), ())),
            preferred_element_type=jnp.float32)


@jax.jit
def kernel(x, Wg, WA, WB):
    bsz, seq, d = x.shape
    T = bsz * seq
    xf = x.reshape(T, d)
    WA_all = WA.reshape(ER, D)
    WB_stack = WB.transpose(0, 2, 1).reshape(ER, D)

    perm = jnp.arange(T, dtype=jnp.int32)
    xg = _sc_gather(xf, perm)

    out = pl.pallas_call(
        _fused_dense_kernel,
        grid=(T // BM,),
        in_specs=[
            pl.BlockSpec((BM, D), lambda i: (i, 0)),
            pl.BlockSpec((E, D), lambda i: (0, 0)),
            pl.BlockSpec((ER, D), lambda i: (0, 0)),
            pl.BlockSpec((ER, D), lambda i: (0, 0)),
        ],
        out_specs=pl.BlockSpec((BM, D), lambda i: (i, 0)),
        out_shape=jax.ShapeDtypeStruct((T, D), jnp.float32),
        compiler_params=pltpu.CompilerParams(dimension_semantics=("parallel",)),
    )(xg, Wg, WA_all, WB_stack)
    return out.reshape(bsz, seq, d)


# speedup vs baseline: 2.2435x; 2.2435x over previous
import jax
import jax.numpy as jnp
from jax.experimental import pallas as pl
from jax.experimental.pallas import tpu as pltpu

E = 8
R = 64
D = 2048
ER = E * R
SCALING = 4.0

BM = 1024
SUB = 2


def _fused_dense_kernel(x_ref, wg_ref, wa_ref, wb_ref, o_ref):
    wg = wg_ref[...]
    wa = wa_ref[...]
    wb = wb_ref[...]
    hb = BM // SUB
    xs = [x_ref[s * hb:(s + 1) * hb, :] for s in range(SUB)]
    logits = [jax.lax.dot_general(x, wg, (((1,), (1,)), ((), ())),
                                  preferred_element_type=jnp.float32)
              for x in xs]
    cmats = []
    for lg in logits:
        maxv = jnp.max(lg, axis=1, keepdims=True)
        denom = jnp.sum(jnp.exp(lg - maxv), axis=1, keepdims=True)
        coef = SCALING / denom
        eidx = jax.lax.broadcasted_iota(jnp.int32, lg.shape, 1)
        gate = jnp.min(jnp.where(lg >= maxv, eidx, E), axis=1, keepdims=True)
        col_e = jax.lax.broadcasted_iota(jnp.int32, (hb, ER), 1) // R
        cmats.append(jnp.where(col_e == gate, coef, 0.0))
    hs = [jax.lax.dot_general(x, wa, (((1,), (1,)), ((), ())),
                              preferred_element_type=jnp.float32)
          for x in xs]
    for s in range(SUB):
        o_ref[s * hb:(s + 1) * hb, :] = jax.lax.dot_general(
            hs[s] * cmats[s], wb, (((1,), (0,)), ((), ())),
            preferred_element_type=jnp.float32)


@jax.jit
def kernel(x, Wg, WA, WB):
    bsz, seq, d = x.shape
    T = bsz * seq
    xf = x.reshape(T, d)
    WA_all = WA.reshape(ER, D)
    WB_stack = WB.transpose(0, 2, 1).reshape(ER, D)

    out = pl.pallas_call(
        _fused_dense_kernel,
        grid=(T // BM,),
        in_specs=[
            pl.BlockSpec((BM, D), lambda i: (i, 0)),
            pl.BlockSpec((E, D), lambda i: (0, 0)),
            pl.BlockSpec((ER, D), lambda i: (0, 0)),
            pl.BlockSpec((ER, D), lambda i: (0, 0)),
        ],
        out_specs=pl.BlockSpec((BM, D), lambda i: (i, 0)),
        out_shape=jax.ShapeDtypeStruct((T, D), jnp.float32),
        compiler_params=pltpu.CompilerParams(dimension_semantics=("parallel",)),
    )(xf, Wg, WA_all, WB_stack)
    return out.reshape(bsz, seq, d)
